# baseline (speedup 1.0000x reference)
import jax
import jax.numpy as jnp
from jax import lax
from jax.experimental import pallas as pl
from jax.experimental.pallas import tpu as pltpu
from jax.experimental.pallas import tpu_sc as plsc

_NC = 2
_NS = 16
_NW = _NC * _NS
_L = 16

_CHUNK = 160


def _sc_add_pe(x_flat, colors_flat, spat_flat, chrom_flat, hw, d):
    n_words = x_flat.shape[0]
    n_rows = n_words // d
    half = d // 2
    rows_per_w = n_rows // _NW
    n_chunks = rows_per_w // _CHUNK
    n_groups = _CHUNK // _L

    mesh = plsc.VectorSubcoreMesh(core_axis_name="c", subcore_axis_name="s")

    @pl.kernel(
        out_type=jax.ShapeDtypeStruct((n_words,), jnp.float32),
        mesh=mesh,
        compiler_params=pltpu.CompilerParams(needs_layout_passes=False),
        scratch_types=[
            pltpu.VMEM((hw * half,), jnp.float32),
            pltpu.VMEM((16 * half,), jnp.float32),
            pltpu.VMEM((rows_per_w,), jnp.int32),
            pltpu.VMEM((_CHUNK * d,), jnp.float32),
        ],
    )
    def body(x_hbm, col_hbm, spat_hbm, chrom_hbm, out_hbm, spat_v, chrom_v, col_v, xbuf):
        wid = lax.axis_index("s") * _NC + lax.axis_index("c")
        base = wid * rows_per_w
        pltpu.sync_copy(spat_hbm, spat_v)
        pltpu.sync_copy(chrom_hbm, chrom_v.at[pl.ds(0, chrom_hbm.shape[0])])
        pltpu.sync_copy(col_hbm.at[pl.ds(base, rows_per_w)], col_v)

        lane = lax.iota(jnp.int32, _L)

        def chunk_body(ci, carry):
            off = ci * _CHUNK
            pltpu.sync_copy(x_hbm.at[pl.ds((base + off) * d, _CHUNK * d)], xbuf)

            def group_body(g, carry2):
                loc = g * _L + lane
                locb = loc * d
                gpix = base + off + loc
                prowb = lax.rem(gpix, hw) * half
                crowb = col_v[pl.ds(off + g * _L, _L)] * half
                for k in range(half):
                    sv = plsc.load_gather(spat_v, [prowb + k])
                    plsc.addupdate_scatter(xbuf, [locb + k], sv)
                    cv = plsc.load_gather(chrom_v, [crowb + k])
                    plsc.addupdate_scatter(xbuf, [locb + (half + k)], cv)
                return carry2

            lax.fori_loop(0, n_groups, group_body, 0, unroll=False)
            pltpu.sync_copy(xbuf, out_hbm.at[pl.ds((base + off) * d, _CHUNK * d)])
            return carry

        lax.fori_loop(0, n_chunks, chunk_body, 0, unroll=False)

    return body(x_flat, colors_flat, spat_flat, chrom_flat)


def kernel(x, color_indices, spatial_pe, chromatic_pe):
    b, h, w, d = x.shape
    half = d // 2
    x_flat = x.reshape(b * h * w * d)
    colors_flat = color_indices.reshape(b * h * w).astype(jnp.int32)
    spat_flat = spatial_pe[:h, :w, :].reshape(h * w * half)
    chrom_flat = chromatic_pe.reshape(-1)
    out = _sc_add_pe(x_flat, colors_flat, spat_flat, chrom_flat, h * w, d)
    return out.reshape(b, h, w, d)


# speedup vs baseline: 2.0885x; 2.0885x over previous
import jax
import jax.numpy as jnp
from jax import lax
from jax.experimental import pallas as pl
from jax.experimental.pallas import tpu as pltpu
from jax.experimental.pallas import tpu_sc as plsc

_NC = 2
_NS = 16
_NW = _NC * _NS
_L = 16

_CHUNK = 160
_UNROLL = 16


def _sc_add_pe(x_flat, colors_flat, spat_flat, chrom_flat, hw, d):
    n_words = x_flat.shape[0]
    n_rows = n_words // d
    half = d // 2
    vecs = half // _L
    rows_per_w = n_rows // _NW
    n_chunks = rows_per_w // _CHUNK
    n_steps = _CHUNK // _UNROLL

    mesh = plsc.VectorSubcoreMesh(core_axis_name="c", subcore_axis_name="s")

    @pl.kernel(
        out_type=jax.ShapeDtypeStruct((n_words,), jnp.float32),
        mesh=mesh,
        compiler_params=pltpu.CompilerParams(needs_layout_passes=False),
        scratch_types=[
            pltpu.VMEM((hw * half,), jnp.float32),
            pltpu.VMEM((16 * half,), jnp.float32),
            pltpu.VMEM((rows_per_w,), jnp.int32),
            pltpu.VMEM((_CHUNK * d,), jnp.float32),
        ],
    )
    def body(x_hbm, col_hbm, spat_hbm, chrom_hbm, out_hbm, spat_v, chrom_v, col_v, xbuf):
        wid = lax.axis_index("s") * _NC + lax.axis_index("c")
        base = wid * rows_per_w
        pltpu.sync_copy(spat_hbm, spat_v)
        pltpu.sync_copy(chrom_hbm, chrom_v.at[pl.ds(0, chrom_hbm.shape[0])])
        pltpu.sync_copy(col_hbm.at[pl.ds(base, rows_per_w)], col_v)

        def chunk_body(ci, carry):
            off = ci * _CHUNK
            pltpu.sync_copy(x_hbm.at[pl.ds((base + off) * d, _CHUNK * d)], xbuf)

            def step_body(s, carry2):
                p0 = s * _UNROLL
                col16 = col_v[pl.ds(off + p0, _UNROLL)]
                for j in range(_UNROLL):
                    p = p0 + j
                    prow = lax.rem(base + off + p, hw)
                    crow = col16[j]
                    sbase = prow * half
                    cbase = crow * half
                    xb = p * d
                    for v in range(vecs):
                        xs = xbuf[pl.ds(xb + v * _L, _L)]
                        sv = spat_v[pl.ds(sbase + v * _L, _L)]
                        xbuf[pl.ds(xb + v * _L, _L)] = xs + sv
                    for v in range(vecs):
                        xc = xbuf[pl.ds(xb + half + v * _L, _L)]
                        cv = chrom_v[pl.ds(cbase + v * _L, _L)]
                        xbuf[pl.ds(xb + half + v * _L, _L)] = xc + cv
                return carry2

            lax.fori_loop(0, n_steps, step_body, 0, unroll=False)
            pltpu.sync_copy(xbuf, out_hbm.at[pl.ds((base + off) * d, _CHUNK * d)])
            return carry

        lax.fori_loop(0, n_chunks, chunk_body, 0, unroll=False)

    return body(x_flat, colors_flat, spat_flat, chrom_flat)


def kernel(x, color_indices, spatial_pe, chromatic_pe):
    b, h, w, d = x.shape
    half = d // 2
    x_flat = x.reshape(b * h * w * d)
    colors_flat = color_indices.reshape(b * h * w).astype(jnp.int32)
    spat_flat = spatial_pe[:h, :w, :].reshape(h * w * half)
    chrom_flat = chromatic_pe.reshape(-1)
    out = _sc_add_pe(x_flat, colors_flat, spat_flat, chrom_flat, h * w, d)
    return out.reshape(b, h, w, d)


# speedup vs baseline: 2.4009x; 1.1496x over previous
import jax
import jax.numpy as jnp
from jax import lax
from jax.experimental import pallas as pl
from jax.experimental.pallas import tpu as pltpu
from jax.experimental.pallas import tpu_sc as plsc

_NC = 2
_NS = 16
_NW = _NC * _NS
_L = 16

_CHUNK = 80
_UNROLL = 16


def _sc_add_pe(x_flat, colors_flat, spat_flat, chrom_flat, hw, d):
    n_words = x_flat.shape[0]
    n_rows = n_words // d
    half = d // 2
    vecs = half // _L
    rows_per_w = n_rows // _NW
    n_chunks = rows_per_w // _CHUNK
    n_steps = _CHUNK // _UNROLL
    cwords = _CHUNK * d

    mesh = plsc.VectorSubcoreMesh(core_axis_name="c", subcore_axis_name="s")

    @pl.kernel(
        out_type=jax.ShapeDtypeStruct((n_words,), jnp.float32),
        mesh=mesh,
        compiler_params=pltpu.CompilerParams(needs_layout_passes=False),
        scratch_types=[
            pltpu.VMEM((hw * half,), jnp.float32),
            pltpu.VMEM((16 * half,), jnp.float32),
            pltpu.VMEM((rows_per_w,), jnp.int32),
            pltpu.VMEM((cwords,), jnp.float32),
            pltpu.VMEM((cwords,), jnp.float32),
            pltpu.VMEM((cwords,), jnp.float32),
            pltpu.VMEM((cwords,), jnp.float32),
            pltpu.SemaphoreType.DMA,
            pltpu.SemaphoreType.DMA,
            pltpu.SemaphoreType.DMA,
            pltpu.SemaphoreType.DMA,
        ],
    )
    def body(x_hbm, col_hbm, spat_hbm, chrom_hbm, out_hbm,
             spat_v, chrom_v, col_v, ib0, ib1, ob0, ob1,
             si0, si1, so0, so1):
        wid = lax.axis_index("s") * _NC + lax.axis_index("c")
        base = wid * rows_per_w
        pltpu.sync_copy(spat_hbm, spat_v)
        pltpu.sync_copy(chrom_hbm, chrom_v.at[pl.ds(0, chrom_hbm.shape[0])])
        pltpu.sync_copy(col_hbm.at[pl.ds(base, rows_per_w)], col_v)

        ibufs, obufs = (ib0, ib1), (ob0, ob1)
        isems, osems = (si0, si1), (so0, so1)

        def start_in(c, b):
            pltpu.async_copy(
                x_hbm.at[pl.ds((base + c * _CHUNK) * d, cwords)], ibufs[b], isems[b])

        def start_out(c, b):
            pltpu.async_copy(
                obufs[b], out_hbm.at[pl.ds((base + c * _CHUNK) * d, cwords)], osems[b])

        def wait_in(b):
            pltpu.make_async_copy(x_hbm.at[pl.ds(0, cwords)], ibufs[b], isems[b]).wait()

        def wait_out(b):
            pltpu.make_async_copy(obufs[b], out_hbm.at[pl.ds(0, cwords)], osems[b]).wait()

        def compute(c, b):
            off = c * _CHUNK
            ib, ob = ibufs[b], obufs[b]

            def step_body(s, carry2):
                p0 = s * _UNROLL
                col16 = col_v[pl.ds(off + p0, _UNROLL)]
                for j in range(_UNROLL):
                    p = p0 + j
                    prow = lax.rem(base + off + p, hw)
                    crow = col16[j]
                    sbase = prow * half
                    cbase = crow * half
                    xb = p * d
                    for v in range(vecs):
                        o = v * _L
                        ob[pl.ds(xb + o, _L)] = ib[pl.ds(xb + o, _L)] + spat_v[pl.ds(sbase + o, _L)]
                    for v in range(vecs):
                        o = v * _L
                        ob[pl.ds(xb + half + o, _L)] = (
                            ib[pl.ds(xb + half + o, _L)] + chrom_v[pl.ds(cbase + o, _L)])
                return carry2

            lax.fori_loop(0, n_steps, step_body, 0, unroll=False)

        start_in(0, 0)
        start_in(1, 1)

        def pair_body(g, carry):
            for b in range(2):
                c = g * 2 + b

                @pl.when(c >= 2)
                def _():
                    wait_out(b)

                wait_in(b)
                compute(c, b)
                start_out(c, b)

                @pl.when(c + 2 < n_chunks)
                def _():
                    start_in(c + 2, b)
            return carry

        lax.fori_loop(0, n_chunks // 2, pair_body, 0, unroll=False)
        wait_out(0)
        wait_out(1)

    return body(x_flat, colors_flat, spat_flat, chrom_flat)


def kernel(x, color_indices, spatial_pe, chromatic_pe):
    b, h, w, d = x.shape
    half = d // 2
    x_flat = x.reshape(b * h * w * d)
    colors_flat = color_indices.reshape(b * h * w).astype(jnp.int32)
    spat_flat = spatial_pe[:h, :w, :].reshape(h * w * half)
    chrom_flat = chromatic_pe.reshape(-1)
    out = _sc_add_pe(x_flat, colors_flat, spat_flat, chrom_flat, h * w, d)
    return out.reshape(b, h, w, d)


# speedup vs baseline: 2.9605x; 1.2331x over previous
import jax
import jax.numpy as jnp
from jax import lax
from jax.experimental import pallas as pl
from jax.experimental.pallas import tpu as pltpu
from jax.experimental.pallas import tpu_sc as plsc

_NC = 2
_NS = 16
_NW = _NC * _NS
_L = 16

_NBUF = 4


def _sc_add_pe(x_lines, colors_flat, spat_flat, chrom_flat, w, hh, d):
    n_lines = x_lines.shape[0]
    half = d // 2
    vecs = half // _L
    lines_per_w = n_lines // _NW
    cols_per_w = lines_per_w * w

    mesh = plsc.VectorSubcoreMesh(core_axis_name="c", subcore_axis_name="s")

    @pl.kernel(
        out_type=jax.ShapeDtypeStruct((n_lines, w, d), jnp.float32),
        mesh=mesh,
        compiler_params=pltpu.CompilerParams(
            needs_layout_passes=False, use_tc_tiling_on_sc=True),
        scratch_types=[
            pltpu.VMEM((hh * w * half,), jnp.float32),
            pltpu.VMEM((16 * half,), jnp.float32),
            pltpu.VMEM((cols_per_w + 16,), jnp.int32),
            pltpu.VMEM((w, d), jnp.float32),
            pltpu.VMEM((w, d), jnp.float32),
            pltpu.VMEM((w, d), jnp.float32),
            pltpu.VMEM((w, d), jnp.float32),
            pltpu.SemaphoreType.DMA,
            pltpu.SemaphoreType.DMA,
            pltpu.SemaphoreType.DMA,
            pltpu.SemaphoreType.DMA,
            pltpu.SemaphoreType.DMA,
            pltpu.SemaphoreType.DMA,
            pltpu.SemaphoreType.DMA,
            pltpu.SemaphoreType.DMA,
        ],
    )
    def body(x_hbm, col_hbm, spat_hbm, chrom_hbm, out_hbm,
             spat_v, chrom_v, col_v, rb0, rb1, rb2, rb3,
             si0, si1, si2, si3, so0, so1, so2, so3):
        wid = lax.axis_index("s") * _NC + lax.axis_index("c")
        base = wid * lines_per_w
        pltpu.sync_copy(spat_hbm, spat_v)
        pltpu.sync_copy(chrom_hbm, chrom_v.at[pl.ds(0, chrom_hbm.shape[0])])
        pltpu.sync_copy(col_hbm.at[pl.ds(wid * cols_per_w, cols_per_w)],
                        col_v.at[pl.ds(0, cols_per_w)])

        bufs = (rb0, rb1, rb2, rb3)
        isems = (si0, si1, si2, si3)
        osems = (so0, so1, so2, so3)

        def start_in(c, b):
            pltpu.async_copy(x_hbm.at[base + c], bufs[b], isems[b])

        def start_out(c, b):
            pltpu.async_copy(bufs[b], out_hbm.at[base + c], osems[b])

        def wait_in(b):
            pltpu.make_async_copy(x_hbm.at[0], bufs[b], isems[b]).wait()

        def wait_out(b):
            pltpu.make_async_copy(bufs[b], out_hbm.at[0], osems[b]).wait()

        def compute(c, b):
            buf = bufs[b]
            line = base + c
            hbase = lax.rem(line, hh) * (w * half)
            coff = c * w
            col16a = col_v[pl.ds(coff, _L)]
            col16b = col_v[pl.ds(coff + _L, _L)]
            for pw in range(w):
                crow = col16a[pw] if pw < _L else col16b[pw - _L]
                sbase = hbase + pw * half
                cbase = crow * half
                for v in range(vecs):
                    o = v * _L
                    buf[pw, pl.ds(o, _L)] = (
                        buf[pw, pl.ds(o, _L)] + spat_v[pl.ds(sbase + o, _L)])
                for v in range(vecs):
                    o = v * _L
                    buf[pw, pl.ds(half + o, _L)] = (
                        buf[pw, pl.ds(half + o, _L)] + chrom_v[pl.ds(cbase + o, _L)])

        start_in(0, 0)
        start_in(1, 1)

        def ring_body(g, carry):
            for b in range(_NBUF):
                c = g * _NBUF + b
                wait_in(b)
                compute(c, b)
                start_out(c, b)

                @pl.when(c >= 2)
                def _():
                    wait_out((b - 2) % _NBUF)

                @pl.when(c + 2 < lines_per_w)
                def _():
                    start_in(c + 2, (b + 2) % _NBUF)
            return carry

        lax.fori_loop(0, lines_per_w // _NBUF, ring_body, 0, unroll=False)
        wait_out((lines_per_w - 2) % _NBUF)
        wait_out((lines_per_w - 1) % _NBUF)

    return body(x_lines, colors_flat, spat_flat, chrom_flat)


def kernel(x, color_indices, spatial_pe, chromatic_pe):
    b, h, w, d = x.shape
    half = d // 2
    x_lines = x.reshape(b * h, w, d)
    colors_flat = color_indices.reshape(b * h * w).astype(jnp.int32)
    spat_flat = spatial_pe[:h, :w, :].reshape(h * w * half)
    chrom_flat = chromatic_pe.reshape(-1)
    out = _sc_add_pe(x_lines, colors_flat, spat_flat, chrom_flat, w, h, d)
    return out.reshape(b, h, w, d)
